# baseline (speedup 1.0000x reference)
import jax
import jax.numpy as jnp
from jax import lax
from jax.experimental import pallas as pl
from jax.experimental.pallas import tpu as pltpu
from jax.experimental.pallas import tpu_sc as plsc

EMB = 64
SCALE = 8.0
B, L = 4096, 200
VOCAB = 100000
NC, NS = 2, 16
NW = NC * NS
BC = B // NW
LB = 4
NSLICE = 5
SL = L // NSLICE
NG = SL // LB
TQ = 4
TBLK = SL // (2 * TQ)


def _make_emb_body(l_off):
    def _emb_body(table, toks, mid, idx_v, rows_v,
                  si0, si1, sg0, sg1, so0, so1):
        wid = lax.axis_index("s") * NC + lax.axis_index("c")
        b0 = wid * BC
        s_idx = (si0, si1)
        s_g = (sg0, sg1)
        s_o = (so0, so1)

        def fire_idx(g, s):
            pltpu.async_copy(toks.at[pl.ds(l_off + g * LB, LB), pl.ds(b0, BC)],
                             idx_v.at[pl.ds(s * LB, LB)], s_idx[s])

        def wait_idx(g, s):
            pltpu.make_async_copy(
                toks.at[pl.ds(l_off + g * LB, LB), pl.ds(b0, BC)],
                idx_v.at[pl.ds(s * LB, LB)], s_idx[s]).wait()

        def run_gathers(s):
            descs = [
                pltpu.async_copy(table.at[idx_v.at[s * LB + j]],
                                 rows_v.at[s * LB + j], s_g[s])
                for j in range(LB)
            ]
            for d in descs:
                d.wait()

        def _out_copies(g, s, fire):
            for j in range(LB):
                q = g * (LB // 2) + j // 2
                dst = mid.at[q, pl.ds(b0, BC), pl.ds((j % 2) * EMB, EMB)]
                if fire:
                    pltpu.async_copy(rows_v.at[s * LB + j], dst, s_o[s])
                else:
                    pltpu.make_async_copy(rows_v.at[s * LB + j], dst,
                                          s_o[s]).wait()

        def fire_out(g, s):
            _out_copies(g, s, True)

        def wait_out(g, s):
            _out_copies(g, s, False)

        fire_idx(0, 0)
        fire_idx(1, 1)
        for g in (0, 1):
            s = g
            wait_idx(g, s)
            run_gathers(s)
            fire_out(g, s)
            fire_idx(g + 2, s)

        @pl.loop(0, (NG - 4) // 2)
        def _steady(i):
            for s in range(2):
                g = 2 + i * 2 + s
                wait_idx(g, s)
                wait_out(g - 2, s)
                run_gathers(s)
                fire_out(g, s)
                fire_idx(g + 2, s)

        for g in (NG - 2, NG - 1):
            s = g % 2
            wait_idx(g, s)
            wait_out(g - 2, s)
            run_gathers(s)
            fire_out(g, s)
        for g in (NG - 2, NG - 1):
            wait_out(g, g % 2)

    return _emb_body


def _emb_gather_slice(table, toks_t, k):
    mesh = plsc.VectorSubcoreMesh(core_axis_name="c", subcore_axis_name="s",
                                  num_cores=NC, num_subcores=NS)
    f = pl.kernel(
        _make_emb_body(k * SL),
        out_type=jax.ShapeDtypeStruct((SL // 2, B, 2 * EMB), jnp.float32),
        mesh=mesh,
        scratch_types=[
            pltpu.VMEM((2 * LB, BC), jnp.int32),
            pltpu.VMEM((2 * LB, BC, EMB), jnp.float32),
            pltpu.SemaphoreType.DMA,
            pltpu.SemaphoreType.DMA,
            pltpu.SemaphoreType.DMA,
            pltpu.SemaphoreType.DMA,
            pltpu.SemaphoreType.DMA,
            pltpu.SemaphoreType.DMA,
        ],
        compiler_params=pltpu.CompilerParams(use_tc_tiling_on_sc=False),
    )
    return f(table, toks_t)


def _tr_blocks(x_ref, o_ref):
    for q in range(TQ):
        t = jnp.transpose(x_ref[q]) * SCALE
        o_ref[2 * q] = t[:EMB]
        o_ref[2 * q + 1] = t[EMB:]


def _tr_body(x_ref, o_ref):
    _tr_blocks(x_ref, o_ref)


def _tr_body_acc(x_ref, prev_ref, o_ref):
    del prev_ref
    _tr_blocks(x_ref, o_ref)


def _transpose_scale_slice(mid_k, k, prev):
    out_shape = jax.ShapeDtypeStruct((L, EMB, B), jnp.float32)
    out_spec = pl.BlockSpec((2 * TQ, EMB, B),
                            lambda i, k=k: (i + k * TBLK, 0, 0))
    in_spec = pl.BlockSpec((TQ, B, 2 * EMB), lambda i: (i, 0, 0))
    if prev is None:
        return pl.pallas_call(
            _tr_body,
            out_shape=out_shape,
            grid=(TBLK,),
            in_specs=[in_spec],
            out_specs=out_spec,
        )(mid_k)
    return pl.pallas_call(
        _tr_body_acc,
        out_shape=out_shape,
        grid=(TBLK,),
        in_specs=[in_spec, pl.BlockSpec(memory_space=pl.ANY)],
        out_specs=out_spec,
        input_output_aliases={1: 0},
    )(mid_k, prev)


def kernel(tokens, table):
    toks_t = jnp.transpose(tokens)
    mids = [_emb_gather_slice(table, toks_t, k) for k in range(NSLICE)]
    out_t = None
    for k in range(NSLICE):
        out_t = _transpose_scale_slice(mids[k], k, out_t)
    return jnp.transpose(out_t, (2, 0, 1))


# speedup vs baseline: 1.0263x; 1.0263x over previous
import jax
import jax.numpy as jnp
from jax import lax
from jax.experimental import pallas as pl
from jax.experimental.pallas import tpu as pltpu
from jax.experimental.pallas import tpu_sc as plsc

EMB = 64
SCALE = 8.0
B, L = 4096, 200
VOCAB = 100000
NC, NS = 2, 16
NW = NC * NS
BC = B // NW
LB = 4
NG = L // LB
TQ = 4


def _emb_body(table, toks, mid, idx_v, rows_v, si0, si1, sg0, sg1, so0, so1):
    wid = lax.axis_index("s") * NC + lax.axis_index("c")
    b0 = wid * BC
    s_idx = (si0, si1)
    s_g = (sg0, sg1)
    s_o = (so0, so1)

    def fire_idx(g, s):
        pltpu.async_copy(toks.at[pl.ds(g * LB, LB), pl.ds(b0, BC)],
                         idx_v.at[pl.ds(s * LB, LB)], s_idx[s])

    def wait_idx(g, s):
        pltpu.make_async_copy(toks.at[pl.ds(g * LB, LB), pl.ds(b0, BC)],
                              idx_v.at[pl.ds(s * LB, LB)], s_idx[s]).wait()

    def run_gathers(s):
        descs = [
            pltpu.async_copy(table.at[idx_v.at[s * LB + j]],
                             rows_v.at[s * LB + j], s_g[s])
            for j in range(LB)
        ]
        for d in descs:
            d.wait()

    def _out_copies(g, s, fire):
        for j in range(LB):
            q = g * (LB // 2) + j // 2
            dst = mid.at[q, pl.ds(b0, BC), pl.ds((j % 2) * EMB, EMB)]
            if fire:
                pltpu.async_copy(rows_v.at[s * LB + j], dst, s_o[s])
            else:
                pltpu.make_async_copy(rows_v.at[s * LB + j], dst,
                                      s_o[s]).wait()

    def fire_out(g, s):
        _out_copies(g, s, True)

    def wait_out(g, s):
        _out_copies(g, s, False)

    fire_idx(0, 0)
    fire_idx(1, 1)
    for g in (0, 1):
        s = g
        wait_idx(g, s)
        run_gathers(s)
        fire_out(g, s)
        fire_idx(g + 2, s)

    @pl.loop(0, (NG - 4) // 2)
    def _steady(i):
        for s in range(2):
            g = 2 + i * 2 + s
            wait_idx(g, s)
            wait_out(g - 2, s)
            run_gathers(s)
            fire_out(g, s)
            fire_idx(g + 2, s)

    for g in (NG - 2, NG - 1):
        s = g % 2
        wait_idx(g, s)
        wait_out(g - 2, s)
        run_gathers(s)
        fire_out(g, s)
    for g in (NG - 2, NG - 1):
        wait_out(g, g % 2)


def _emb_gather(table, toks_t):
    mesh = plsc.VectorSubcoreMesh(core_axis_name="c", subcore_axis_name="s",
                                  num_cores=NC, num_subcores=NS)
    f = pl.kernel(
        _emb_body,
        out_type=jax.ShapeDtypeStruct((L // 2, B, 2 * EMB), jnp.float32),
        mesh=mesh,
        scratch_types=[
            pltpu.VMEM((2 * LB, BC), jnp.int32),
            pltpu.VMEM((2 * LB, BC, EMB), jnp.float32),
            pltpu.SemaphoreType.DMA,
            pltpu.SemaphoreType.DMA,
            pltpu.SemaphoreType.DMA,
            pltpu.SemaphoreType.DMA,
            pltpu.SemaphoreType.DMA,
            pltpu.SemaphoreType.DMA,
        ],
        compiler_params=pltpu.CompilerParams(use_tc_tiling_on_sc=False),
    )
    return f(table, toks_t)


def _tr_body(x_ref, o_ref):
    for q in range(TQ):
        t = jnp.transpose(x_ref[q]) * SCALE
        o_ref[2 * q] = t[:EMB]
        o_ref[2 * q + 1] = t[EMB:]


def _transpose_scale(mid):
    return pl.pallas_call(
        _tr_body,
        out_shape=jax.ShapeDtypeStruct((L, EMB, B), jnp.float32),
        grid=(L // (2 * TQ),),
        in_specs=[pl.BlockSpec((TQ, B, 2 * EMB), lambda i: (i, 0, 0))],
        out_specs=pl.BlockSpec((2 * TQ, EMB, B), lambda i: (i, 0, 0)),
    )(mid)


def kernel(tokens, table):
    toks_t = jnp.transpose(tokens)
    mid = _emb_gather(table, toks_t)
    out_t = _transpose_scale(mid)
    return jnp.transpose(out_t, (2, 0, 1))
